# baseline (speedup 1.0000x reference)
import jax
import jax.numpy as jnp
from jax.experimental import pallas as pl
from jax.experimental.pallas import tpu as pltpu

IMG = 512
KK = 1000
CC = 0.001 / KK
CH = 2
NBUF = 8
NCHUNK = 128 // CH


def _body(steps_ref, x_hbm, o_hbm, ibufs, obufs, sem_in, sem_out, eye_ref):
    c = pl.program_id(0)
    slot = jax.lax.rem(c, NBUF)

    def load(chunk, slt):
        return pltpu.make_async_copy(
            x_hbm.at[pl.ds(chunk * CH, CH)], ibufs.at[slt], sem_in.at[slt])

    def store(chunk, slt):
        return pltpu.make_async_copy(
            obufs.at[slt], o_hbm.at[pl.ds(chunk * CH, CH)], sem_out.at[slt])

    @pl.when(c == 0)
    def _prologue():
        rows = jax.lax.broadcasted_iota(jnp.int32, (IMG, IMG), 0)
        cols = jax.lax.broadcasted_iota(jnp.int32, (IMG, IMG), 1)
        eye_ref[...] = jnp.where(rows == cols, 1.0, 0.0).astype(jnp.float32)
        for b in range(NBUF):
            load(b, b).start()

    load(c, slot).wait()

    @pl.when(c >= NBUF)
    def _wait_prev_store():
        store(c - NBUF, slot).wait()

    eye = eye_ref[...]
    for s in range(CH):
        step = steps_ref[c * CH + s]
        idx = jnp.where(step == 0, KK - 1, step - 1)
        val = CC * idx.astype(jnp.float32)
        obufs[slot, s] = ibufs[slot, s] + val * eye

    store(c, slot).start()

    @pl.when(c + NBUF < NCHUNK)
    def _next_load():
        load(c + NBUF, slot).start()

    @pl.when(c == NCHUNK - 1)
    def _epilogue():
        for b in range(NBUF):
            store(0, b).wait()


def kernel(x, fwd_steps):
    grid_spec = pltpu.PrefetchScalarGridSpec(
        num_scalar_prefetch=1,
        grid=(NCHUNK,),
        in_specs=[pl.BlockSpec(memory_space=pl.ANY)],
        out_specs=pl.BlockSpec(memory_space=pl.ANY),
        scratch_shapes=[
            pltpu.VMEM((NBUF, CH, IMG, IMG), jnp.float32),
            pltpu.VMEM((NBUF, CH, IMG, IMG), jnp.float32),
            pltpu.SemaphoreType.DMA((NBUF,)),
            pltpu.SemaphoreType.DMA((NBUF,)),
            pltpu.VMEM((IMG, IMG), jnp.float32),
        ],
    )
    return pl.pallas_call(
        _body,
        grid_spec=grid_spec,
        out_shape=jax.ShapeDtypeStruct(x.shape, x.dtype),
    )(fwd_steps.astype(jnp.int32), x)


# speedup vs baseline: 1.0008x; 1.0008x over previous
import jax
import jax.numpy as jnp
from jax.experimental import pallas as pl
from jax.experimental.pallas import tpu as pltpu

IMG = 512
KK = 1000
CC = 0.001 / KK
CH = 1
NBUF = 8
NCHUNK = 128 // CH


def _body(steps_ref, x_hbm, o_hbm, ibufs, obufs, sem_in, sem_out, eye_ref):
    c = pl.program_id(0)
    slot = jax.lax.rem(c, NBUF)

    def load(chunk, slt):
        return pltpu.make_async_copy(
            x_hbm.at[pl.ds(chunk * CH, CH)], ibufs.at[slt], sem_in.at[slt])

    def store(chunk, slt):
        return pltpu.make_async_copy(
            obufs.at[slt], o_hbm.at[pl.ds(chunk * CH, CH)], sem_out.at[slt])

    @pl.when(c == 0)
    def _prologue():
        rows = jax.lax.broadcasted_iota(jnp.int32, (IMG, IMG), 0)
        cols = jax.lax.broadcasted_iota(jnp.int32, (IMG, IMG), 1)
        eye_ref[...] = jnp.where(rows == cols, 1.0, 0.0).astype(jnp.float32)
        for b in range(NBUF):
            load(b, b).start()

    load(c, slot).wait()

    @pl.when(c >= NBUF)
    def _wait_prev_store():
        store(c - NBUF, slot).wait()

    eye = eye_ref[...]
    for s in range(CH):
        step = steps_ref[c * CH + s]
        idx = jnp.where(step == 0, KK - 1, step - 1)
        val = CC * idx.astype(jnp.float32)
        obufs[slot, s] = ibufs[slot, s] + val * eye

    store(c, slot).start()

    @pl.when(c + NBUF < NCHUNK)
    def _next_load():
        load(c + NBUF, slot).start()

    @pl.when(c == NCHUNK - 1)
    def _epilogue():
        for b in range(NBUF):
            store(0, b).wait()


def kernel(x, fwd_steps):
    grid_spec = pltpu.PrefetchScalarGridSpec(
        num_scalar_prefetch=1,
        grid=(NCHUNK,),
        in_specs=[pl.BlockSpec(memory_space=pl.ANY)],
        out_specs=pl.BlockSpec(memory_space=pl.ANY),
        scratch_shapes=[
            pltpu.VMEM((NBUF, CH, IMG, IMG), jnp.float32),
            pltpu.VMEM((NBUF, CH, IMG, IMG), jnp.float32),
            pltpu.SemaphoreType.DMA((NBUF,)),
            pltpu.SemaphoreType.DMA((NBUF,)),
            pltpu.VMEM((IMG, IMG), jnp.float32),
        ],
    )
    return pl.pallas_call(
        _body,
        grid_spec=grid_spec,
        out_shape=jax.ShapeDtypeStruct(x.shape, x.dtype),
    )(fwd_steps.astype(jnp.int32), x)


# speedup vs baseline: 1.0013x; 1.0005x over previous
import jax
import jax.numpy as jnp
from jax.experimental import pallas as pl
from jax.experimental.pallas import tpu as pltpu

IMG = 512
KK = 1000
CC = 0.001 / KK
CH = 2
NBUF = 8
NCHUNK = 128 // CH


def _body(steps_ref, x_hbm, o_hbm, ibufs, obufs, sem_in, sem_out, eye_ref):
    c = pl.program_id(0)
    slot = jax.lax.rem(c, NBUF)

    def load(chunk, slt):
        return pltpu.make_async_copy(
            x_hbm.at[pl.ds(chunk * CH, CH)], ibufs.at[slt], sem_in.at[slt])

    def store(chunk, slt):
        return pltpu.make_async_copy(
            obufs.at[slt], o_hbm.at[pl.ds(chunk * CH, CH)], sem_out.at[slt])

    @pl.when(c == 0)
    def _prologue():
        rows = jax.lax.broadcasted_iota(jnp.int32, (IMG, IMG), 0)
        cols = jax.lax.broadcasted_iota(jnp.int32, (IMG, IMG), 1)
        eye_ref[...] = jnp.where(rows == cols, 1.0, 0.0).astype(jnp.float32)
        for b in range(NBUF):
            load(b, b).start()

    load(c, slot).wait()

    @pl.when(c >= NBUF)
    def _wait_prev_store():
        store(c - NBUF, slot).wait()

    eye = eye_ref[...]
    for s in range(CH):
        step = steps_ref[c * CH + s]
        idx = jnp.where(step == 0, KK - 1, step - 1)
        val = CC * idx.astype(jnp.float32)
        obufs[slot, s] = ibufs[slot, s] + val * eye

    store(c, slot).start()

    @pl.when(c + NBUF < NCHUNK)
    def _next_load():
        load(c + NBUF, slot).start()

    @pl.when(c == NCHUNK - 1)
    def _epilogue():
        for b in range(NBUF):
            store(0, b).wait()


def kernel(x, fwd_steps):
    grid_spec = pltpu.PrefetchScalarGridSpec(
        num_scalar_prefetch=1,
        grid=(NCHUNK,),
        in_specs=[pl.BlockSpec(memory_space=pl.ANY)],
        out_specs=pl.BlockSpec(memory_space=pl.ANY),
        scratch_shapes=[
            pltpu.VMEM((NBUF, CH, IMG, IMG), jnp.float32),
            pltpu.VMEM((NBUF, CH, IMG, IMG), jnp.float32),
            pltpu.SemaphoreType.DMA((NBUF,)),
            pltpu.SemaphoreType.DMA((NBUF,)),
            pltpu.VMEM((IMG, IMG), jnp.float32),
        ],
    )
    return pl.pallas_call(
        _body,
        grid_spec=grid_spec,
        out_shape=jax.ShapeDtypeStruct(x.shape, x.dtype),
    )(fwd_steps.astype(jnp.int32), x)
